# baseline (speedup 1.0000x reference)
import functools

import jax
import jax.numpy as jnp
from jax import lax
from jax.experimental import pallas as pl
from jax.experimental.pallas import tpu as pltpu
from jax.experimental.pallas import tpu_sc as plsc

B_TOK = 16384
SEQ = 26
NUM_ROWS = 1000000
DIM = 64

NC = 2
NS = 16
NW = NC * NS
CH = 128
TB = B_TOK // CH
NBLK = SEQ * TB
BPW = NBLK // NW
NBUF = 4


def _build_gather():
    mesh = plsc.VectorSubcoreMesh(core_axis_name="c", subcore_axis_name="s")

    @functools.partial(
        pl.kernel,
        out_type=jax.ShapeDtypeStruct((SEQ * DIM * B_TOK,), jnp.float32),
        mesh=mesh,
        compiler_params=pltpu.CompilerParams(needs_layout_passes=False),
        scratch_types=[
            pltpu.VMEM((BPW, CH), jnp.int32),
            *[pltpu.VMEM((CH, CH), jnp.float32) for _ in range(NBUF)],
            *[pltpu.VMEM((DIM * CH,), jnp.float32) for _ in range(NBUF)],
            pltpu.SemaphoreType.DMA,
            pltpu.SemaphoreType.DMA,
        ],
    )
    def grab(table_hbm, idx_hbm, out_hbm, idx_v, *rest):
        rows_bufs = rest[:NBUF]
        tp_bufs = rest[NBUF : 2 * NBUF]
        gsem, ssem = rest[2 * NBUF], rest[2 * NBUF + 1]

        wid = lax.axis_index("s") * NC + lax.axis_index("c")
        blk0 = wid * BPW
        pltpu.sync_copy(idx_hbm.at[pl.ds(blk0, BPW)], idx_v)

        lane = lax.iota(jnp.int32, 16)
        rotk = [(lane + k) & 15 for k in range(16)]
        scatk = [((lane + k) & 15) * CH + lane for k in range(16)]

        def fire(j, b):
            pltpu.async_copy(table_hbm.at[idx_v.at[j]], rows_bufs[b], gsem)

        def store_copies(j, b):
            s = (blk0 + j) // TB
            tb = (blk0 + j) % TB
            return [
                pltpu.make_async_copy(
                    tp_bufs[b].at[pl.ds(dt * (8 * CH), 8 * CH)],
                    out_hbm.at[pl.ds(((s * 8 + dt) * TB + tb) * (8 * CH), 8 * CH)],
                    ssem,
                )
                for dt in range(DIM // 8)
            ]

        def store(j, b):
            for c in store_copies(j, b):
                c.start()

        def wait_store(j, b):
            for c in store_copies(j, b):
                c.wait()

        for b in range(NBUF):
            fire(b, b)

        def group(g, carry):
            for b in range(NBUF):
                j = g * NBUF + b
                pltpu.make_async_copy(
                    table_hbm.at[idx_v.at[j]], rows_bufs[b], gsem
                ).wait()

                @pl.when(j >= NBUF)
                def _drain():
                    wait_store(j - NBUF, b)

                src = rows_bufs[b]
                dst = tp_bufs[b]

                def trans(i, c):
                    t0 = i * 16
                    rowv = lane + t0
                    for d0 in range(0, DIM, 16):
                        for k in range(16):
                            v = plsc.load_gather(src, [rowv, rotk[k] + d0])
                            plsc.store_scatter(
                                dst, [(scatk[k] + d0 * CH) + t0], v
                            )
                    return c

                lax.fori_loop(0, CH // 16, trans, 0)

                store(j, b)

                @pl.when(j + NBUF < BPW)
                def _fire_next():
                    fire(j + NBUF, b)

            return carry

        lax.fori_loop(0, BPW // NBUF, group, 0)

        for b in range(NBUF):
            wait_store(BPW - NBUF + b, b)

    return grab


VBLK = 2048
NGRID = (NUM_ROWS + VBLK - 1) // VBLK


def _tc_pairs(dt_ref, out_ref):
    out_ref[:, :DIM] = dt_ref[...].T


def _build_pairs():
    return pl.pallas_call(
        _tc_pairs,
        grid=(NGRID,),
        in_specs=[pl.BlockSpec((DIM, VBLK), lambda i: (0, i))],
        out_specs=pl.BlockSpec((VBLK, 2 * DIM), lambda i: (i, 0)),
        out_shape=jax.ShapeDtypeStruct((NUM_ROWS, 2 * DIM), jnp.float32),
    )


def kernel(token_ids, embedding_table):
    idx = token_ids.astype(jnp.int32).T.reshape(NBLK, CH)
    table2 = _build_pairs()(embedding_table.T)
    flat = _build_gather()(table2, idx)
    out5 = flat.reshape(SEQ, DIM // 8, TB, 8, CH)
    return out5.transpose(2, 4, 0, 1, 3).reshape(B_TOK, SEQ, DIM)


# speedup vs baseline: 1.0216x; 1.0216x over previous
import functools

import jax
import jax.numpy as jnp
from jax import lax
from jax.experimental import pallas as pl
from jax.experimental.pallas import tpu as pltpu
from jax.experimental.pallas import tpu_sc as plsc

B_TOK = 16384
SEQ = 26
NUM_ROWS = 1000000
DIM = 64

NC = 2
NS = 16
NW = NC * NS
CH = 128
TB = B_TOK // CH
NBLK = SEQ * TB
BPW = NBLK // NW
NBUF = 2


def _build_gather():
    mesh = plsc.VectorSubcoreMesh(core_axis_name="c", subcore_axis_name="s")

    @functools.partial(
        pl.kernel,
        out_type=jax.ShapeDtypeStruct((SEQ * DIM * B_TOK,), jnp.float32),
        mesh=mesh,
        compiler_params=pltpu.CompilerParams(needs_layout_passes=False),
        scratch_types=[
            pltpu.VMEM((BPW, CH), jnp.int32),
            *[pltpu.VMEM((CH, CH), jnp.float32) for _ in range(NBUF)],
            *[pltpu.VMEM((DIM * CH,), jnp.float32) for _ in range(NBUF)],
            pltpu.SemaphoreType.DMA,
            pltpu.SemaphoreType.DMA,
        ],
    )
    def grab(table_hbm, idx_hbm, out_hbm, idx_v, *rest):
        rows_bufs = rest[:NBUF]
        tp_bufs = rest[NBUF : 2 * NBUF]
        gsem, ssem = rest[2 * NBUF], rest[2 * NBUF + 1]

        wid = lax.axis_index("s") * NC + lax.axis_index("c")
        blk0 = wid * BPW
        pltpu.sync_copy(idx_hbm.at[pl.ds(blk0, BPW)], idx_v)

        lane = lax.iota(jnp.int32, 16)
        rotk = [(lane + k) & 15 for k in range(16)]
        scatk = [((lane + k) & 15) * CH + lane for k in range(16)]

        def fire(j, b):
            pltpu.async_copy(table_hbm.at[idx_v.at[j]], rows_bufs[b], gsem)

        def store_copies(j, b):
            s = (blk0 + j) // TB
            tb = (blk0 + j) % TB
            return [
                pltpu.make_async_copy(
                    tp_bufs[b].at[pl.ds(dt * (8 * CH), 8 * CH)],
                    out_hbm.at[pl.ds(((s * 8 + dt) * TB + tb) * (8 * CH), 8 * CH)],
                    ssem,
                )
                for dt in range(DIM // 8)
            ]

        def store(j, b):
            for c in store_copies(j, b):
                c.start()

        def wait_store(j, b):
            for c in store_copies(j, b):
                c.wait()

        for b in range(NBUF):
            fire(b, b)

        def group(g, carry):
            for b in range(NBUF):
                j = g * NBUF + b
                pltpu.make_async_copy(
                    table_hbm.at[idx_v.at[j]], rows_bufs[b], gsem
                ).wait()

                @pl.when(j >= NBUF)
                def _drain():
                    wait_store(j - NBUF, b)

                src = rows_bufs[b]
                dst = tp_bufs[b]

                def trans(i, c):
                    t0 = i * 16
                    rowv = lane + t0
                    for d0 in range(0, DIM, 16):
                        for k in range(16):
                            v = plsc.load_gather(src, [rowv, rotk[k] + d0])
                            plsc.store_scatter(
                                dst, [(scatk[k] + d0 * CH) + t0], v
                            )
                    return c

                lax.fori_loop(0, CH // 16, trans, 0)

                store(j, b)

                @pl.when(j + NBUF < BPW)
                def _fire_next():
                    fire(j + NBUF, b)

            return carry

        lax.fori_loop(0, BPW // NBUF, group, 0)

        for b in range(NBUF):
            wait_store(BPW - NBUF + b, b)

    return grab


VBLK = 2048
NGRID = (NUM_ROWS + VBLK - 1) // VBLK


def _tc_pairs(dt_ref, out_ref):
    out_ref[:, :DIM] = dt_ref[...].T


def _build_pairs():
    return pl.pallas_call(
        _tc_pairs,
        grid=(NGRID,),
        in_specs=[pl.BlockSpec((DIM, VBLK), lambda i: (0, i))],
        out_specs=pl.BlockSpec((VBLK, 2 * DIM), lambda i: (i, 0)),
        out_shape=jax.ShapeDtypeStruct((NUM_ROWS, 2 * DIM), jnp.float32),
    )


def kernel(token_ids, embedding_table):
    idx = token_ids.astype(jnp.int32).T.reshape(NBLK, CH)
    table2 = _build_pairs()(embedding_table.T)
    flat = _build_gather()(table2, idx)
    out5 = flat.reshape(SEQ, DIM // 8, TB, 8, CH)
    return out5.transpose(2, 4, 0, 1, 3).reshape(B_TOK, SEQ, DIM)


# speedup vs baseline: 1.2147x; 1.1890x over previous
import functools

import jax
import jax.numpy as jnp
from jax import lax
from jax.experimental import pallas as pl
from jax.experimental.pallas import tpu as pltpu
from jax.experimental.pallas import tpu_sc as plsc

B_TOK = 16384
SEQ = 26
NUM_ROWS = 1000000
DIM = 64

NC = 2
NS = 16
NW = NC * NS
CH = 128
TB = B_TOK // CH
NBLK = SEQ * TB
BPW = NBLK // NW
NBUF = 2


def _build_gather():
    mesh = plsc.VectorSubcoreMesh(core_axis_name="c", subcore_axis_name="s")

    @functools.partial(
        pl.kernel,
        out_type=jax.ShapeDtypeStruct((SEQ * DIM * B_TOK,), jnp.float32),
        mesh=mesh,
        compiler_params=pltpu.CompilerParams(needs_layout_passes=False),
        scratch_types=[
            pltpu.VMEM((BPW, CH), jnp.int32),
            *[pltpu.VMEM((CH, CH), jnp.float32) for _ in range(NBUF)],
            *[pltpu.VMEM((DIM * CH,), jnp.float32) for _ in range(NBUF)],
            pltpu.SemaphoreType.DMA,
            pltpu.SemaphoreType.DMA,
        ],
    )
    def grab(table_hbm, idx_hbm, out_hbm, idx_v, *rest):
        rows_bufs = rest[:NBUF]
        tp_bufs = rest[NBUF : 2 * NBUF]
        gsem, ssem = rest[2 * NBUF], rest[2 * NBUF + 1]

        wid = lax.axis_index("s") * NC + lax.axis_index("c")
        blk0 = wid * BPW
        pltpu.sync_copy(idx_hbm.at[pl.ds(blk0, BPW)], idx_v)

        lane = lax.iota(jnp.int32, 16)
        rotk = [(lane + k) & 15 for k in range(16)]
        scatk = [((lane + k) & 15) * CH + lane for k in range(16)]

        def fire(j, b):
            pltpu.async_copy(table_hbm.at[idx_v.at[j]], rows_bufs[b], gsem)

        def store_copies(j, b):
            s = (blk0 + j) // TB
            tb = (blk0 + j) % TB
            return [
                pltpu.make_async_copy(
                    tp_bufs[b].at[pl.ds(dt * (8 * CH), 8 * CH)],
                    out_hbm.at[pl.ds(((s * 8 + dt) * TB + tb) * (8 * CH), 8 * CH)],
                    ssem,
                )
                for dt in range(DIM // 8)
            ]

        def store(j, b):
            for c in store_copies(j, b):
                c.start()

        def wait_store(j, b):
            for c in store_copies(j, b):
                c.wait()

        for b in range(NBUF):
            fire(b, b)

        def group(g, carry):
            for b in range(NBUF):
                j = g * NBUF + b
                pltpu.make_async_copy(
                    table_hbm.at[idx_v.at[j]], rows_bufs[b], gsem
                ).wait()

                @pl.when(j >= NBUF)
                def _drain():
                    wait_store(j - NBUF, b)

                src = rows_bufs[b]
                dst = tp_bufs[b]

                def trans(i, c):
                    t0 = i * 16
                    rowv = lane + t0
                    for d0 in range(0, DIM, 16):
                        for k in range(16):
                            v = plsc.load_gather(src, [rowv, rotk[k] + d0])
                            plsc.store_scatter(
                                dst, [(scatk[k] + d0 * CH) + t0], v
                            )
                    return c

                lax.fori_loop(0, CH // 16, trans, 0)

                store(j, b)

                @pl.when(j + NBUF < BPW)
                def _fire_next():
                    fire(j + NBUF, b)

            return carry

        lax.fori_loop(0, BPW // NBUF, group, 0)

        for b in range(NBUF):
            wait_store(BPW - NBUF + b, b)

    return grab


VBLK = 4096
NGRID = (NUM_ROWS + VBLK - 1) // VBLK


def _tc_pairs(dt_ref, out_ref):
    out_ref[:, :DIM] = dt_ref[...].T


def _build_pairs():
    return pl.pallas_call(
        _tc_pairs,
        grid=(NGRID,),
        in_specs=[pl.BlockSpec((DIM, VBLK), lambda i: (0, i))],
        out_specs=pl.BlockSpec((VBLK, 2 * DIM), lambda i: (i, 0)),
        out_shape=jax.ShapeDtypeStruct((NUM_ROWS, 2 * DIM), jnp.float32),
    )


def kernel(token_ids, embedding_table):
    idx = token_ids.astype(jnp.int32).T.reshape(NBLK, CH)
    table2 = _build_pairs()(embedding_table.T)
    flat = _build_gather()(table2, idx)
    out5 = flat.reshape(SEQ, DIM // 8, TB, 8, CH)
    return out5.transpose(2, 4, 0, 1, 3).reshape(B_TOK, SEQ, DIM)


# speedup vs baseline: 1.3656x; 1.1243x over previous
import functools

import jax
import jax.numpy as jnp
from jax import lax
from jax.experimental import pallas as pl
from jax.experimental.pallas import tpu as pltpu
from jax.experimental.pallas import tpu_sc as plsc

B_TOK = 16384
SEQ = 26
NUM_ROWS = 1000000
DIM = 64

NC = 2
NS = 16
NW = NC * NS
CH = 128
TB = B_TOK // CH
NBLK = SEQ * TB
BPW = NBLK // NW
NBUF = 2


def _build_gather():
    mesh = plsc.VectorSubcoreMesh(core_axis_name="c", subcore_axis_name="s")

    @functools.partial(
        pl.kernel,
        out_type=jax.ShapeDtypeStruct((SEQ * DIM * B_TOK,), jnp.float32),
        mesh=mesh,
        compiler_params=pltpu.CompilerParams(needs_layout_passes=False),
        scratch_types=[
            pltpu.VMEM((BPW, CH), jnp.int32),
            *[pltpu.VMEM((CH, CH), jnp.float32) for _ in range(NBUF)],
            *[pltpu.VMEM((DIM * CH,), jnp.float32) for _ in range(NBUF)],
            pltpu.SemaphoreType.DMA,
            pltpu.SemaphoreType.DMA,
        ],
    )
    def grab(table_hbm, idx_hbm, out_hbm, idx_v, *rest):
        rows_bufs = rest[:NBUF]
        tp_bufs = rest[NBUF : 2 * NBUF]
        gsem, ssem = rest[2 * NBUF], rest[2 * NBUF + 1]

        wid = lax.axis_index("s") * NC + lax.axis_index("c")
        blk0 = wid * BPW
        pltpu.sync_copy(idx_hbm.at[pl.ds(blk0, BPW)], idx_v)

        lane = lax.iota(jnp.int32, 16)
        rotk = [(lane + k) & 15 for k in range(16)]
        scatk = [((lane + k) & 15) * CH + lane for k in range(16)]

        def fire(j, b):
            pltpu.async_copy(table_hbm.at[idx_v.at[j]], rows_bufs[b], gsem)

        def store_copies(j, b):
            s = (blk0 + j) // TB
            tb = (blk0 + j) % TB
            return [
                pltpu.make_async_copy(
                    tp_bufs[b].at[pl.ds(dt * (8 * CH), 8 * CH)],
                    out_hbm.at[pl.ds(((s * 8 + dt) * TB + tb) * (8 * CH), 8 * CH)],
                    ssem,
                )
                for dt in range(DIM // 8)
            ]

        def store(j, b):
            for c in store_copies(j, b):
                c.start()

        def wait_store(j, b):
            for c in store_copies(j, b):
                c.wait()

        for b in range(NBUF):
            fire(b, b)

        def group(g, carry):
            for b in range(NBUF):
                j = g * NBUF + b
                pltpu.make_async_copy(
                    table_hbm.at[idx_v.at[j]], rows_bufs[b], gsem
                ).wait()

                @pl.when(j >= NBUF)
                def _drain():
                    wait_store(j - NBUF, b)

                src = rows_bufs[b]
                dst = tp_bufs[b]

                def trans(i, c):
                    t0 = i * 16
                    rowv = lane + t0
                    for d0 in range(0, DIM, 16):
                        for k in range(16):
                            v = plsc.load_gather(src, [rowv, rotk[k] + d0])
                            plsc.store_scatter(
                                dst, [(scatk[k] + d0 * CH) + t0], v
                            )
                    return c

                lax.fori_loop(0, CH // 16, trans, 0)

                store(j, b)

                @pl.when(j + NBUF < BPW)
                def _fire_next():
                    fire(j + NBUF, b)

            return carry

        lax.fori_loop(0, BPW // NBUF, group, 0)

        for b in range(NBUF):
            wait_store(BPW - NBUF + b, b)

    return grab


VBLK = 8192
NGRID = (NUM_ROWS + VBLK - 1) // VBLK


def _tc_pairs(dt_ref, out_ref):
    out_ref[:, :DIM] = dt_ref[...].T


def _build_pairs():
    return pl.pallas_call(
        _tc_pairs,
        grid=(NGRID,),
        in_specs=[pl.BlockSpec((DIM, VBLK), lambda i: (0, i))],
        out_specs=pl.BlockSpec((VBLK, 2 * DIM), lambda i: (i, 0)),
        out_shape=jax.ShapeDtypeStruct((NUM_ROWS, 2 * DIM), jnp.float32),
    )


def kernel(token_ids, embedding_table):
    idx = token_ids.astype(jnp.int32).T.reshape(NBLK, CH)
    table2 = _build_pairs()(embedding_table.T)
    flat = _build_gather()(table2, idx)
    out5 = flat.reshape(SEQ, DIM // 8, TB, 8, CH)
    return out5.transpose(2, 4, 0, 1, 3).reshape(B_TOK, SEQ, DIM)
